# baseline (speedup 1.0000x reference)
import functools

import jax
import jax.numpy as jnp
from jax import lax
from jax.experimental import pallas as pl
from jax.experimental.pallas import tpu as pltpu
from jax.experimental.pallas import tpu_sc as plsc

_VROW = 104
_LANE = 128
_INV = 1.0 / (1.0 + 1e-5) ** 0.5
_NC, _NS, _L = 2, 16, 16


def _sc_first_order(ids_hbm, vals_hbm, fo_hbm, out_hbm,
                    ids_v, vals_v, acc_v, fo_v, *, f, bpw):
    wid = lax.axis_index("s") * _NC + lax.axis_index("c")
    base = wid * bpw
    pltpu.sync_copy(fo_hbm, fo_v)
    pltpu.sync_copy(ids_hbm.at[:, pl.ds(base, bpw)], ids_v)
    pltpu.sync_copy(vals_hbm.at[:, pl.ds(base, bpw)], vals_v)
    nj = bpw // _L
    for j in range(nj):
        acc_v[pl.ds(j * _L, _L)] = jnp.zeros((_L,), jnp.float32)

    def body(fi, carry):
        for j in range(nj):
            sl = pl.ds(j * _L, _L)
            idx = ids_v[fi, sl]
            fo16 = plsc.load_gather(fo_v, [idx])
            acc_v[sl] = acc_v[sl] + fo16 * vals_v[fi, sl]
        return carry

    lax.fori_loop(0, f, body, 0)
    pltpu.sync_copy(acc_v, out_hbm.at[pl.ds(base, bpw)])


def _t2_body(so_ref, w1_ref, out_ref):
    out_ref[...] = jnp.dot(so_ref[...], w1_ref[...],
                           preferred_element_type=jnp.float32
                           ).astype(jnp.bfloat16)


def _tc_body(idsb_ref, tab32_ref, t2_ref, w2_ref, w3t_ref,
             b1_ref, s1_ref, t1_ref, b2_ref, s2_ref, t2b_ref, b3_ref,
             out_ref, oht_ref, *, tb, f, d):
    viota = lax.broadcasted_iota(jnp.int32, (_VROW, tb), 0).astype(jnp.bfloat16)
    ids_bf = idsb_ref[...].astype(jnp.bfloat16)
    counts = jnp.zeros((_VROW, tb), jnp.bfloat16)
    for fi in range(f):
        idr = jnp.broadcast_to(ids_bf[fi:fi + 1, :], (_VROW, tb))
        ohf = jnp.where(idr == viota, jnp.bfloat16(1), jnp.bfloat16(0))
        oht_ref[fi * _VROW:(fi + 1) * _VROW, :] = ohf
        counts = counts + ohf

    h = lax.dot_general(oht_ref[...], t2_ref[...], (((0,), (0,)), ((), ())),
                        preferred_element_type=jnp.float32)
    h = jnp.maximum(h + b1_ref[...], 0.0) * s1_ref[...] + t1_ref[...]
    h = jnp.dot(h.astype(jnp.bfloat16), w2_ref[...],
                preferred_element_type=jnp.float32)
    h = jnp.maximum(h + b2_ref[...], 0.0) * s2_ref[...] + t2b_ref[...]
    deep = jnp.sum(h * w3t_ref[...], axis=1, keepdims=True) + b3_ref[...]

    esum = lax.dot_general(counts.astype(jnp.float32), tab32_ref[...],
                           (((0,), (0,)), ((), ())),
                           preferred_element_type=jnp.float32)
    lane = lax.broadcasted_iota(jnp.int32, (tb, _LANE), 1)
    sq = jnp.where(lane < d, esum * esum, 0.0)
    square_sum = jnp.sum(sq, axis=1, keepdims=True)
    sum_square = esum[:, d + 1:d + 2]
    second = 0.5 * (square_sum - sum_square)

    out_ref[...] = (second + deep).reshape(1, tb // _LANE, _LANE)


def _comb_body(a_ref, b_ref, o_ref):
    logit = a_ref[...] + b_ref[...]
    o_ref[...] = 1.0 / (1.0 + jnp.exp(-logit))


def kernel(feature_ids, feature_values, fo_table, so_table,
           W1, b1, g1, be1, W2, b2, g2, be2, W3, b3):
    b, f = feature_ids.shape
    v, d = so_table.shape
    h1 = W1.shape[1]
    h2 = W2.shape[1]
    tb = 512 if b % 512 == 0 else b
    grid = b // tb
    ids32 = feature_ids.astype(jnp.int32)
    ids_t = ids32.T

    nw = _NC * _NS
    bpw = b // nw
    fo_pad = jnp.zeros((_LANE,), jnp.float32).at[:v].set(fo_table[:, 0])
    sc_fn = functools.partial(_sc_first_order, f=f, bpw=bpw)
    first = pl.kernel(
        sc_fn,
        out_type=jax.ShapeDtypeStruct((b,), jnp.float32),
        mesh=plsc.VectorSubcoreMesh(core_axis_name="c", subcore_axis_name="s",
                                    num_cores=_NC, num_subcores=_NS),
        compiler_params=pltpu.CompilerParams(needs_layout_passes=False),
        scratch_types=[
            pltpu.VMEM((f, bpw), jnp.int32),
            pltpu.VMEM((f, bpw), jnp.float32),
            pltpu.VMEM((bpw,), jnp.float32),
            pltpu.VMEM((_LANE,), jnp.float32),
        ],
    )(ids_t, feature_values.T, fo_pad)

    so_pad = jnp.zeros((_VROW, d), jnp.float32).at[:v, :].set(so_table)
    t2 = pl.pallas_call(
        _t2_body,
        grid=(f,),
        in_specs=[
            pl.BlockSpec((_VROW, d), lambda i: (0, 0)),
            pl.BlockSpec((d, h1), lambda i: (i, 0)),
        ],
        out_specs=pl.BlockSpec((_VROW, h1), lambda i: (i, 0)),
        out_shape=jax.ShapeDtypeStruct((f * _VROW, h1), jnp.bfloat16),
    )(so_pad, W1)

    rn2 = jnp.sum(so_table * so_table, axis=1)
    tab32 = jnp.zeros((_VROW, _LANE), jnp.float32)
    tab32 = tab32.at[:v, :d].set(so_table)
    tab32 = tab32.at[:v, d].set(fo_table[:, 0])
    tab32 = tab32.at[:v, d + 1].set(rn2)

    s1 = (g1 * _INV).reshape(1, h1)
    s2 = (g2 * _INV).reshape(1, h2)

    body = functools.partial(_tc_body, tb=tb, f=f, d=d)
    full = lambda i: (0, 0)
    dense = pl.pallas_call(
        body,
        grid=(grid,),
        in_specs=[
            pl.BlockSpec((f, tb), lambda i: (0, i)),
            pl.BlockSpec((_VROW, _LANE), full),
            pl.BlockSpec((f * _VROW, h1), full),
            pl.BlockSpec((h1, h2), full),
            pl.BlockSpec((1, h2), full),
            pl.BlockSpec((1, h1), full),
            pl.BlockSpec((1, h1), full),
            pl.BlockSpec((1, h1), full),
            pl.BlockSpec((1, h2), full),
            pl.BlockSpec((1, h2), full),
            pl.BlockSpec((1, h2), full),
            pl.BlockSpec((1, 1), full),
        ],
        out_specs=pl.BlockSpec((1, tb // _LANE, _LANE), lambda i: (i, 0, 0)),
        out_shape=jax.ShapeDtypeStruct((grid, tb // _LANE, _LANE),
                                       jnp.float32),
        scratch_shapes=[
            pltpu.VMEM((f * _VROW, tb), jnp.bfloat16),
        ],
    )(ids_t, tab32, t2,
      W2.astype(jnp.bfloat16), W3.reshape(1, h2),
      b1.reshape(1, h1), s1, be1.reshape(1, h1),
      b2.reshape(1, h2), s2, be2.reshape(1, h2), b3.reshape(1, 1))

    rows = b // _LANE
    out = pl.pallas_call(
        _comb_body,
        out_shape=jax.ShapeDtypeStruct((rows, _LANE), jnp.float32),
    )(dense.reshape(rows, _LANE), first.reshape(rows, _LANE))
    return out.reshape(b)


# speedup vs baseline: 1.1925x; 1.1925x over previous
import functools

import jax
import jax.numpy as jnp
from jax import lax
from jax.experimental import pallas as pl
from jax.experimental.pallas import tpu as pltpu
from jax.experimental.pallas import tpu_sc as plsc

_VROW = 104
_LANE = 128
_INV = 1.0 / (1.0 + 1e-5) ** 0.5
_NC, _NS, _L = 2, 16, 16


def _sc_first_order(ids_hbm, vals_hbm, fo_hbm, out_hbm,
                    ids_v, vals_v, acc_v, fo_v, *, f, bpw):
    wid = lax.axis_index("s") * _NC + lax.axis_index("c")
    base = wid * bpw
    pltpu.sync_copy(fo_hbm, fo_v)
    pltpu.sync_copy(ids_hbm.at[:, pl.ds(base, bpw)], ids_v)
    pltpu.sync_copy(vals_hbm.at[:, pl.ds(base, bpw)], vals_v)
    nj = bpw // _L
    for j in range(nj):
        acc_v[pl.ds(j * _L, _L)] = jnp.zeros((_L,), jnp.float32)

    def body(fi, carry):
        for j in range(nj):
            sl = pl.ds(j * _L, _L)
            idx = ids_v[fi, sl]
            fo16 = plsc.load_gather(fo_v, [idx])
            acc_v[sl] = acc_v[sl] + fo16 * vals_v[fi, sl]
        return carry

    lax.fori_loop(0, f, body, 0)
    pltpu.sync_copy(acc_v, out_hbm.at[pl.ds(base, bpw)])


def _tc_body(idsb_ref, tab32_ref, sob_ref, w1_ref, w2_ref, w3t_ref,
             b1_ref, s1_ref, t1_ref, b2_ref, s2_ref, t2b_ref, b3_ref,
             out_ref, oht_ref, t2_ref, *, tb, f, d):
    @pl.when(pl.program_id(0) == 0)
    def _build_t2():
        for fi in range(f):
            t2_ref[fi * _VROW:(fi + 1) * _VROW, :] = jnp.dot(
                sob_ref[...], w1_ref[fi * d:(fi + 1) * d, :],
                preferred_element_type=jnp.float32).astype(jnp.bfloat16)

    viota = lax.broadcasted_iota(jnp.int32, (_VROW, tb), 0).astype(jnp.bfloat16)
    ids_bf = idsb_ref[...].astype(jnp.bfloat16)
    counts = jnp.zeros((_VROW, tb), jnp.bfloat16)
    for fi in range(f):
        idr = jnp.broadcast_to(ids_bf[fi:fi + 1, :], (_VROW, tb))
        ohf = jnp.where(idr == viota, jnp.bfloat16(1), jnp.bfloat16(0))
        oht_ref[fi * _VROW:(fi + 1) * _VROW, :] = ohf
        counts = counts + ohf

    h = lax.dot_general(oht_ref[...], t2_ref[...], (((0,), (0,)), ((), ())),
                        preferred_element_type=jnp.float32)
    h = jnp.maximum(h + b1_ref[...], 0.0) * s1_ref[...] + t1_ref[...]
    h = jnp.dot(h.astype(jnp.bfloat16), w2_ref[...],
                preferred_element_type=jnp.float32)
    h = jnp.maximum(h + b2_ref[...], 0.0) * s2_ref[...] + t2b_ref[...]
    deep = jnp.sum(h * w3t_ref[...], axis=1, keepdims=True) + b3_ref[...]

    esum = lax.dot_general(counts.astype(jnp.float32), tab32_ref[...],
                           (((0,), (0,)), ((), ())),
                           preferred_element_type=jnp.float32)
    lane = lax.broadcasted_iota(jnp.int32, (tb, _LANE), 1)
    sq = jnp.where(lane < d, esum * esum, 0.0)
    square_sum = jnp.sum(sq, axis=1, keepdims=True)
    sum_square = esum[:, d + 1:d + 2]
    second = 0.5 * (square_sum - sum_square)

    out_ref[...] = (second + deep).reshape(1, tb // _LANE, _LANE)


def _comb_body(a_ref, b_ref, o_ref):
    logit = a_ref[...] + b_ref[...]
    o_ref[...] = 1.0 / (1.0 + jnp.exp(-logit))


def kernel(feature_ids, feature_values, fo_table, so_table,
           W1, b1, g1, be1, W2, b2, g2, be2, W3, b3):
    b, f = feature_ids.shape
    v, d = so_table.shape
    h1 = W1.shape[1]
    h2 = W2.shape[1]
    tb = 512 if b % 512 == 0 else b
    grid = b // tb
    ids32 = feature_ids.astype(jnp.int32)
    ids_t = ids32.T

    nw = _NC * _NS
    bpw = b // nw
    fo_pad = jnp.zeros((_LANE,), jnp.float32).at[:v].set(fo_table[:, 0])
    sc_fn = functools.partial(_sc_first_order, f=f, bpw=bpw)
    first = pl.kernel(
        sc_fn,
        out_type=jax.ShapeDtypeStruct((b,), jnp.float32),
        mesh=plsc.VectorSubcoreMesh(core_axis_name="c", subcore_axis_name="s",
                                    num_cores=_NC, num_subcores=_NS),
        compiler_params=pltpu.CompilerParams(needs_layout_passes=False),
        scratch_types=[
            pltpu.VMEM((f, bpw), jnp.int32),
            pltpu.VMEM((f, bpw), jnp.float32),
            pltpu.VMEM((bpw,), jnp.float32),
            pltpu.VMEM((_LANE,), jnp.float32),
        ],
    )(ids_t, feature_values.T, fo_pad)

    so_pad = jnp.zeros((_VROW, d), jnp.float32).at[:v, :].set(so_table)

    rn2 = jnp.sum(so_table * so_table, axis=1)
    tab32 = jnp.zeros((_VROW, _LANE), jnp.float32)
    tab32 = tab32.at[:v, :d].set(so_table)
    tab32 = tab32.at[:v, d].set(fo_table[:, 0])
    tab32 = tab32.at[:v, d + 1].set(rn2)

    s1 = (g1 * _INV).reshape(1, h1)
    s2 = (g2 * _INV).reshape(1, h2)

    body = functools.partial(_tc_body, tb=tb, f=f, d=d)
    full = lambda i: (0, 0)
    dense = pl.pallas_call(
        body,
        grid=(grid,),
        in_specs=[
            pl.BlockSpec((f, tb), lambda i: (0, i)),
            pl.BlockSpec((_VROW, _LANE), full),
            pl.BlockSpec((_VROW, d), full),
            pl.BlockSpec((f * d, h1), full),
            pl.BlockSpec((h1, h2), full),
            pl.BlockSpec((1, h2), full),
            pl.BlockSpec((1, h1), full),
            pl.BlockSpec((1, h1), full),
            pl.BlockSpec((1, h1), full),
            pl.BlockSpec((1, h2), full),
            pl.BlockSpec((1, h2), full),
            pl.BlockSpec((1, h2), full),
            pl.BlockSpec((1, 1), full),
        ],
        out_specs=pl.BlockSpec((1, tb // _LANE, _LANE), lambda i: (i, 0, 0)),
        out_shape=jax.ShapeDtypeStruct((grid, tb // _LANE, _LANE),
                                       jnp.float32),
        scratch_shapes=[
            pltpu.VMEM((f * _VROW, tb), jnp.bfloat16),
            pltpu.VMEM((f * _VROW, h1), jnp.bfloat16),
        ],
    )(ids_t, tab32, so_pad.astype(jnp.bfloat16), W1.astype(jnp.bfloat16),
      W2.astype(jnp.bfloat16), W3.reshape(1, h2),
      b1.reshape(1, h1), s1, be1.reshape(1, h1),
      b2.reshape(1, h2), s2, be2.reshape(1, h2), b3.reshape(1, 1))

    rows = b // _LANE
    out = pl.pallas_call(
        _comb_body,
        out_shape=jax.ShapeDtypeStruct((rows, _LANE), jnp.float32),
    )(dense.reshape(rows, _LANE), first.reshape(rows, _LANE))
    return out.reshape(b)
